# baseline (speedup 1.0000x reference)
import functools

import jax
import jax.numpy as jnp
import numpy as np
from jax import lax
from jax.experimental import pallas as pl
from jax.experimental.pallas import tpu as pltpu
from jax.experimental.pallas import tpu_sc as plsc

POOL = 7
LANES = 16
NUM_CORES = 2
NUM_SUBCORES = 16
NUM_WORKERS = NUM_CORES * NUM_SUBCORES


def _take16(v, idx):
    return lax.gather(
        v,
        idx[:, None],
        dimension_numbers=lax.GatherDimensionNumbers(
            offset_dims=(), collapsed_slice_dims=(0,), start_index_map=(0,)),
        slice_sizes=(1,),
        mode=lax.GatherScatterMode.PROMISE_IN_BOUNDS,
    )


def _roi_pool_body(n_rois, h, w, rois_per_worker,
                   table, roisp, out,
                   rois_v, idx_p0, idx_p1, patch0, patch1,
                   off_a, off_b, off_c, off_d, out_v0, out_v1,
                   sem0, sem1, osem0, osem1):
    wid = lax.axis_index("s") * NUM_CORES + lax.axis_index("c")
    base = wid * rois_per_worker
    pltpu.sync_copy(roisp.at[pl.ds(base, rois_per_worker + 8)], rois_v)

    io = lax.iota(jnp.int32, LANES)
    sel1 = io >> 3
    pf = (io & 7).astype(jnp.float32)
    limf = jnp.where(io < 8, float(w), float(h))
    limi = jnp.where(io < 8, w, h)

    def bounds(r):
        roi = rois_v[r]
        f1 = jnp.clip(_take16(roi, sel1) / 16.0, 0.0, limf - 1.0)
        f2 = jnp.clip(_take16(roi, sel1 + 2) / 16.0, f1 + 1.0, limf)
        bsz = (f2 - f1) / float(POOL)
        sf = f1 + pf * bsz
        ef = f1 + (pf + 1.0) * bsz
        s = jnp.maximum(sf.astype(jnp.int32), 0)
        ei = ef.astype(jnp.int32)
        e = jnp.minimum(jnp.where(ef > ei.astype(jnp.float32), ei + 1, ei),
                        limi)
        return s, e - 1

    def issue(r, idx_ref, patch_ref, sem):
        roi = rois_v[r]
        f1 = jnp.clip(_take16(roi, sel1) / 16.0, 0.0, limf - 1.0)
        s = jnp.maximum(f1.astype(jnp.int32), 0)
        i0 = io * 0
        sy0 = _take16(s, i0 + 8)
        sx0 = _take16(s, i0)
        for v in range(4):
            k = io + 16 * v
            pi = k >> 3
            pj = k & 7
            idx_ref[pl.ds(v * 16, 16)] = (
                jnp.minimum(sy0 + pi, h - 1) * w
                + jnp.minimum(sx0 + pj, w - 1))
        pltpu.async_copy(table.at[idx_ref], patch_ref, sem)

    def compute(r, idx_ref, patch_ref, sem, out_ref, osem):
        n = base + r

        @pl.when(r >= 2)
        def _():
            pltpu.make_async_copy(out_ref, out.at[n - 2], osem).wait()

        s, b = bounds(r)
        i0 = io * 0
        base8 = _take16(s, i0 + 8) * 8 + _take16(s, i0)
        for v in range(4):
            k = io + 16 * v
            inb = k < POOL * POOL
            p_ = (k * 37) >> 8
            q_ = k - p_ * POOL
            py = jnp.where(inb, p_, 0) + 8
            qx = jnp.where(inb, q_, 0)
            ya = _take16(s, py) * 8
            yb = _take16(b, py) * 8
            xa = _take16(s, qx)
            xb = _take16(b, qx)
            off_a[pl.ds(v * 16, 16)] = ya + xa - base8
            off_b[pl.ds(v * 16, 16)] = ya + xb - base8
            off_c[pl.ds(v * 16, 16)] = yb + xa - base8
            off_d[pl.ds(v * 16, 16)] = yb + xb - base8
        pltpu.make_async_copy(table.at[idx_ref], patch_ref, sem).wait()

        def bin_body(kk, c2):
            ra = off_a[pl.ds(kk, 16)][0]
            rb = off_b[pl.ds(kk, 16)][0]
            rc = off_c[pl.ds(kk, 16)][0]
            rd = off_d[pl.ds(kk, 16)][0]
            for c in range(0, 256, LANES):
                sl = pl.ds(c, LANES)
                m1 = jnp.maximum(patch_ref[ra, sl], patch_ref[rb, sl])
                m2 = jnp.maximum(patch_ref[rc, sl], patch_ref[rd, sl])
                out_ref[kk, sl] = jnp.maximum(m1, m2)
            return c2

        lax.fori_loop(0, POOL * POOL, bin_body, 0)
        pltpu.async_copy(out_ref, out.at[n], osem)

    issue(0, idx_p0, patch0, sem0)

    def pipe_body(g, carry):
        r0 = g * 2
        issue(r0 + 1, idx_p1, patch1, sem1)
        compute(r0, idx_p0, patch0, sem0, out_v0, osem0)
        issue(r0 + 2, idx_p0, patch0, sem0)
        compute(r0 + 1, idx_p1, patch1, sem1, out_v1, osem1)
        return carry

    lax.fori_loop(0, rois_per_worker // 2, pipe_body, 0)

    pltpu.make_async_copy(table.at[idx_p0], patch0, sem0).wait()
    pltpu.make_async_copy(
        out_v0, out.at[base + rois_per_worker - 2], osem0).wait()
    pltpu.make_async_copy(
        out_v1, out.at[base + rois_per_worker - 1], osem1).wait()


@functools.lru_cache(maxsize=None)
def _build(n_rois, h, w, c):
    n_pad = -(-n_rois // NUM_WORKERS) * NUM_WORKERS
    rois_per_worker = n_pad // NUM_WORKERS
    mesh = plsc.VectorSubcoreMesh(core_axis_name="c", subcore_axis_name="s")
    body = functools.partial(_roi_pool_body, n_rois, h, w, rois_per_worker)
    return pl.kernel(
        body,
        mesh=mesh,
        out_type=jax.ShapeDtypeStruct((n_pad, POOL * POOL, c), jnp.float32),
        scratch_types=[
            pltpu.VMEM((rois_per_worker + 8, LANES), jnp.float32),
            pltpu.VMEM((64,), jnp.int32),
            pltpu.VMEM((64,), jnp.int32),
            pltpu.VMEM((64, 256), jnp.float32),
            pltpu.VMEM((64, 256), jnp.float32),
            pltpu.VMEM((64,), jnp.int32),
            pltpu.VMEM((64,), jnp.int32),
            pltpu.VMEM((64,), jnp.int32),
            pltpu.VMEM((64,), jnp.int32),
            pltpu.VMEM((POOL * POOL, 256), jnp.float32),
            pltpu.VMEM((POOL * POOL, 256), jnp.float32),
            pltpu.SemaphoreType.DMA,
            pltpu.SemaphoreType.DMA,
            pltpu.SemaphoreType.DMA,
            pltpu.SemaphoreType.DMA,
        ],
    ), n_pad


def kernel(feat_map, rois):
    b, h, w, c = feat_map.shape
    n = rois.shape[1]
    fn, n_pad = _build(n, h, w, c)
    table = feat_map.reshape(h * w, c)
    roisp = jnp.zeros((n_pad + 8, LANES), jnp.float32).at[:n, :4].set(
        rois.reshape(n, 4))
    out = fn(table, roisp)
    return out[:n].reshape(b, n, POOL, POOL, c)


# speedup vs baseline: 1.0245x; 1.0245x over previous
import functools

import jax
import jax.numpy as jnp
import numpy as np
from jax import lax
from jax.experimental import pallas as pl
from jax.experimental.pallas import tpu as pltpu
from jax.experimental.pallas import tpu_sc as plsc

POOL = 7
LANES = 16
NUM_CORES = 2
NUM_SUBCORES = 16
NUM_WORKERS = NUM_CORES * NUM_SUBCORES


def _take16(v, idx):
    return lax.gather(
        v,
        idx[:, None],
        dimension_numbers=lax.GatherDimensionNumbers(
            offset_dims=(), collapsed_slice_dims=(0,), start_index_map=(0,)),
        slice_sizes=(1,),
        mode=lax.GatherScatterMode.PROMISE_IN_BOUNDS,
    )


def _roi_pool_body(n_rois, h, w, rois_per_worker,
                   table, roisp, out,
                   rois_v, idx2, patch2, out2, gsem, osem):
    wid = lax.axis_index("s") * NUM_CORES + lax.axis_index("c")
    base = wid * rois_per_worker
    pltpu.sync_copy(roisp.at[pl.ds(base, rois_per_worker + 8)], rois_v)

    io = lax.iota(jnp.int32, LANES)
    sel1 = io >> 3
    pf = (io & 7).astype(jnp.float32)
    limf = jnp.where(io < 8, float(w), float(h))
    limi = jnp.where(io < 8, w, h)

    def bounds(r):
        roi = rois_v[r]
        f1 = jnp.clip(_take16(roi, sel1) / 16.0, 0.0, limf - 1.0)
        f2 = jnp.clip(_take16(roi, sel1 + 2) / 16.0, f1 + 1.0, limf)
        bsz = (f2 - f1) / float(POOL)
        sf = f1 + pf * bsz
        ef = f1 + (pf + 1.0) * bsz
        s = jnp.maximum(sf.astype(jnp.int32), 0)
        ei = ef.astype(jnp.int32)
        e = jnp.minimum(jnp.where(ef > ei.astype(jnp.float32), ei + 1, ei),
                        limi)
        return s, e - 1

    def issue(r, slot):
        roi = rois_v[r]
        f1 = jnp.clip(_take16(roi, sel1) / 16.0, 0.0, limf - 1.0)
        s = jnp.maximum(f1.astype(jnp.int32), 0)
        i0 = io * 0
        sy0 = _take16(s, i0 + 8)
        sx0 = _take16(s, i0)
        for v in range(4):
            k = io + 16 * v
            idx2[slot, pl.ds(v * 16, 16)] = (
                jnp.minimum(sy0 + (k >> 3), h - 1) * w
                + jnp.minimum(sx0 + (k & 7), w - 1))
        pltpu.async_copy(table.at[idx2.at[slot]], patch2.at[slot],
                         gsem.at[slot])

    def compute(r, slot, oslot):
        n = base + r

        @pl.when(jnp.logical_and(r >= 2, n - 2 < n_rois))
        def _():
            pltpu.make_async_copy(out2.at[oslot], out.at[n - 2],
                                  osem.at[oslot]).wait()

        s_vec, b_vec = bounds(r)
        sx = [s_vec[q] for q in range(POOL)]
        bx = [b_vec[q] for q in range(POOL)]
        sy = [s_vec[8 + p] for p in range(POOL)]
        by = [b_vec[8 + p] for p in range(POOL)]
        sx0 = sx[0]
        sy0 = sy[0]
        pltpu.make_async_copy(table.at[idx2.at[slot]], patch2.at[slot],
                              gsem.at[slot]).wait()
        for p in range(POOL):
            ry = (sy[p] - sy0) * 8
            yb = (by[p] - sy0) * 8
            for q in range(POOL):
                rx = sx[q] - sx0
                xb = bx[q] - sx0
                ra = ry + rx
                rb = ry + xb
                rc = yb + rx
                rd = yb + xb
                kbin = p * POOL + q

                def cgroup(c4, cc, ra=ra, rb=rb, rc=rc, rd=rd, kbin=kbin):
                    co = c4 * 64
                    for u in range(4):
                        sl = pl.ds(co + u * 16, 16)
                        m1 = jnp.maximum(patch2[slot, ra, sl],
                                         patch2[slot, rb, sl])
                        m2 = jnp.maximum(patch2[slot, rc, sl],
                                         patch2[slot, rd, sl])
                        out2[oslot, kbin, sl] = jnp.maximum(m1, m2)
                    return cc

                lax.fori_loop(0, 4, cgroup, 0)
        @pl.when(n < n_rois)
        def _():
            pltpu.async_copy(out2.at[oslot], out.at[n], osem.at[oslot])

    issue(0, 0)
    issue(1, 1)

    def pipe_body(r, carry):
        issue(r + 2, (r + 2) & 3)
        compute(r, r & 3, r & 1)
        return carry

    lax.fori_loop(0, rois_per_worker, pipe_body, 0)

    rpw = rois_per_worker
    pltpu.make_async_copy(table.at[idx2.at[rpw & 3]], patch2.at[rpw & 3],
                          gsem.at[rpw & 3]).wait()
    pltpu.make_async_copy(table.at[idx2.at[(rpw + 1) & 3]],
                          patch2.at[(rpw + 1) & 3],
                          gsem.at[(rpw + 1) & 3]).wait()
    @pl.when(base + rpw - 2 < n_rois)
    def _():
        pltpu.make_async_copy(out2.at[0], out.at[base + rpw - 2],
                              osem.at[0]).wait()

    @pl.when(base + rpw - 1 < n_rois)
    def _():
        pltpu.make_async_copy(out2.at[1], out.at[base + rpw - 1],
                              osem.at[1]).wait()


@functools.lru_cache(maxsize=None)
def _build(n_rois, h, w, c):
    n_pad = -(-n_rois // NUM_WORKERS) * NUM_WORKERS
    rois_per_worker = n_pad // NUM_WORKERS
    mesh = plsc.VectorSubcoreMesh(core_axis_name="c", subcore_axis_name="s")
    body = functools.partial(_roi_pool_body, n_rois, h, w, rois_per_worker)
    return pl.kernel(
        body,
        mesh=mesh,
        out_type=jax.ShapeDtypeStruct((n_rois, POOL * POOL, c), jnp.float32),
        scratch_types=[
            pltpu.VMEM((rois_per_worker + 8, LANES), jnp.float32),
            pltpu.VMEM((4, 64), jnp.int32),
            pltpu.VMEM((4, 64, 256), jnp.float32),
            pltpu.VMEM((2, POOL * POOL, 256), jnp.float32),
            pltpu.SemaphoreType.DMA((4,)),
            pltpu.SemaphoreType.DMA((2,)),
        ],
    ), n_pad


def kernel(feat_map, rois):
    b, h, w, c = feat_map.shape
    n = rois.shape[1]
    fn, n_pad = _build(n, h, w, c)
    table = feat_map.reshape(h * w, c)
    roisp = jnp.zeros((n_pad + 8, LANES), jnp.float32).at[:n, :4].set(
        rois.reshape(n, 4))
    out = fn(table, roisp)
    return out.reshape(b, n, POOL, POOL, c)


# speedup vs baseline: 1.6792x; 1.6391x over previous
import functools

import jax
import jax.numpy as jnp
import numpy as np
from jax import lax
from jax.experimental import pallas as pl
from jax.experimental.pallas import tpu as pltpu
from jax.experimental.pallas import tpu_sc as plsc

POOL = 7
LANES = 16
NUM_CORES = 2
NUM_SUBCORES = 16
NUM_WORKERS = NUM_CORES * NUM_SUBCORES


def _take16(v, idx):
    return lax.gather(
        v,
        idx[:, None],
        dimension_numbers=lax.GatherDimensionNumbers(
            offset_dims=(), collapsed_slice_dims=(0,), start_index_map=(0,)),
        slice_sizes=(1,),
        mode=lax.GatherScatterMode.PROMISE_IN_BOUNDS,
    )


def _roi_pool_body(n_rois, h, w, rois_per_worker,
                   table, roisp, out,
                   rois_v, idx_p0, idx_p1, patch0, patch1,
                   off_a, off_b, off_c, off_d, out_v0, out_v1,
                   sem0, sem1, osem0, osem1):
    wid = lax.axis_index("s") * NUM_CORES + lax.axis_index("c")
    base = wid * rois_per_worker
    pltpu.sync_copy(roisp.at[pl.ds(base, rois_per_worker)], rois_v)

    io = lax.iota(jnp.int32, LANES)
    sel1 = io >> 3
    pf = (io & 7).astype(jnp.float32)
    limf = jnp.where(io < 8, float(w), float(h))
    limi = jnp.where(io < 8, w, h)

    def bounds(r):
        roi = rois_v[r]
        f1 = jnp.clip(_take16(roi, sel1) / 16.0, 0.0, limf - 1.0)
        f2 = jnp.clip(_take16(roi, sel1 + 2) / 16.0, f1 + 1.0, limf)
        bsz = (f2 - f1) / float(POOL)
        sf = f1 + pf * bsz
        ef = f1 + (pf + 1.0) * bsz
        s = jnp.maximum(sf.astype(jnp.int32), 0)
        ei = ef.astype(jnp.int32)
        e = jnp.minimum(jnp.where(ef > ei.astype(jnp.float32), ei + 1, ei),
                        limi)
        return s, e - 1

    def issue(gp, idx_ref, patch_ref, sem):
        @pl.when(jnp.logical_and(2 * gp < rois_per_worker,
                                 base + 2 * gp < n_rois))
        def _():
            for half in range(2):
                s, _b = bounds(2 * gp + half)
                i0 = io * 0
                sy0 = _take16(s, i0 + 8)
                sx0 = _take16(s, i0)
                for v in range(4):
                    k = io + 16 * v
                    pi = k >> 3
                    pj = k & 7
                    idx_ref[pl.ds(half * 64 + v * 16, 16)] = (
                        jnp.minimum(sy0 + pi, h - 1) * w
                        + jnp.minimum(sx0 + pj, w - 1))
            pltpu.async_copy(table.at[idx_ref], patch_ref, sem)

    def compute(gp, idx_ref, patch_ref, sem, out_ref, osem):
        n = base + 2 * gp

        @pl.when(jnp.logical_and(gp >= 2, n - 4 < n_rois))
        def _():
            pltpu.make_async_copy(out_ref, out.at[pl.ds(n - 4, 2)],
                                  osem).wait()

        @pl.when(n < n_rois)
        def _():
            pltpu.make_async_copy(table.at[idx_ref], patch_ref, sem).wait()
            for half in range(2):
                s, b = bounds(2 * gp + half)
                i0 = io * 0
                base8 = (_take16(s, i0 + 8) * 8 + _take16(s, i0)
                         - half * 64)
                for v in range(4):
                    k = io + 16 * v
                    inb = k < POOL * POOL
                    p_ = (k * 37) >> 8
                    q_ = k - p_ * POOL
                    py = jnp.where(inb, p_, 0) + 8
                    qx = jnp.where(inb, q_, 0)
                    ya = _take16(s, py) * 8
                    yb = _take16(b, py) * 8
                    xa = _take16(s, qx)
                    xb = _take16(b, qx)
                    off_a[pl.ds(v * 16, 16)] = ya + xa - base8
                    off_b[pl.ds(v * 16, 16)] = ya + xb - base8
                    off_c[pl.ds(v * 16, 16)] = yb + xa - base8
                    off_d[pl.ds(v * 16, 16)] = yb + xb - base8

                def bin_body(kk, c2, half=half):
                    ra = off_a[pl.ds(kk, 16)][0]
                    rb = off_b[pl.ds(kk, 16)][0]
                    rc = off_c[pl.ds(kk, 16)][0]
                    rd = off_d[pl.ds(kk, 16)][0]
                    for c in range(0, 256, LANES):
                        sl = pl.ds(c, LANES)
                        m1 = jnp.maximum(patch_ref[ra, sl],
                                         patch_ref[rb, sl])
                        m2 = jnp.maximum(patch_ref[rc, sl],
                                         patch_ref[rd, sl])
                        out_ref[half, kk, sl] = jnp.maximum(m1, m2)
                    return c2

                lax.fori_loop(0, POOL * POOL, bin_body, 0)
            pltpu.async_copy(out_ref, out.at[pl.ds(n, 2)], osem)

    issue(0, idx_p0, patch0, sem0)
    n_pairs = rois_per_worker // 2

    def pipe_body(m, carry):
        g0 = m * 2
        issue(g0 + 1, idx_p1, patch1, sem1)
        compute(g0, idx_p0, patch0, sem0, out_v0, osem0)
        issue(g0 + 2, idx_p0, patch0, sem0)
        compute(g0 + 1, idx_p1, patch1, sem1, out_v1, osem1)
        return carry

    lax.fori_loop(0, n_pairs // 2, pipe_body, 0)

    n0 = base + 2 * (n_pairs - 2)
    n1 = base + 2 * (n_pairs - 1)

    @pl.when(n0 < n_rois)
    def _():
        pltpu.make_async_copy(out_v0, out.at[pl.ds(n0, 2)], osem0).wait()

    @pl.when(n1 < n_rois)
    def _():
        pltpu.make_async_copy(out_v1, out.at[pl.ds(n1, 2)], osem1).wait()


@functools.lru_cache(maxsize=None)
def _build(n_rois, h, w, c):
    n_pad = -(-n_rois // NUM_WORKERS) * NUM_WORKERS
    rois_per_worker = n_pad // NUM_WORKERS
    mesh = plsc.VectorSubcoreMesh(core_axis_name="c", subcore_axis_name="s")
    body = functools.partial(_roi_pool_body, n_rois, h, w, rois_per_worker)
    return pl.kernel(
        body,
        mesh=mesh,
        out_type=jax.ShapeDtypeStruct((n_rois, POOL * POOL, c), jnp.float32),
        scratch_types=[
            pltpu.VMEM((rois_per_worker, LANES), jnp.float32),
            pltpu.VMEM((128,), jnp.int32),
            pltpu.VMEM((128,), jnp.int32),
            pltpu.VMEM((128, 256), jnp.float32),
            pltpu.VMEM((128, 256), jnp.float32),
            pltpu.VMEM((64,), jnp.int32),
            pltpu.VMEM((64,), jnp.int32),
            pltpu.VMEM((64,), jnp.int32),
            pltpu.VMEM((64,), jnp.int32),
            pltpu.VMEM((2, POOL * POOL, 256), jnp.float32),
            pltpu.VMEM((2, POOL * POOL, 256), jnp.float32),
            pltpu.SemaphoreType.DMA,
            pltpu.SemaphoreType.DMA,
            pltpu.SemaphoreType.DMA,
            pltpu.SemaphoreType.DMA,
        ],
    ), n_pad


def kernel(feat_map, rois):
    b, h, w, c = feat_map.shape
    n = rois.shape[1]
    fn, n_pad = _build(n, h, w, c)
    table = feat_map.reshape(h * w, c)
    roisp = jnp.zeros((n_pad, LANES), jnp.float32).at[:n, :4].set(
        rois.reshape(n, 4))
    out = fn(table, roisp)
    return out.reshape(b, n, POOL, POOL, c)


# speedup vs baseline: 1.6976x; 1.0109x over previous
import functools

import jax
import jax.numpy as jnp
import numpy as np
from jax import lax
from jax.experimental import pallas as pl
from jax.experimental.pallas import tpu as pltpu
from jax.experimental.pallas import tpu_sc as plsc

POOL = 7
LANES = 16
NUM_CORES = 2
NUM_SUBCORES = 16
NUM_WORKERS = NUM_CORES * NUM_SUBCORES


def _take16(v, idx):
    return lax.gather(
        v,
        idx[:, None],
        dimension_numbers=lax.GatherDimensionNumbers(
            offset_dims=(), collapsed_slice_dims=(0,), start_index_map=(0,)),
        slice_sizes=(1,),
        mode=lax.GatherScatterMode.PROMISE_IN_BOUNDS,
    )


def _roi_pool_body(n_rois, h, w, rois_per_worker,
                   table, roisp, out,
                   rois_v, idx_p0, idx_p1, patch0, patch1,
                   off_all, out_v0, out_v1,
                   sem0, sem1, osem0, osem1):
    wid = lax.axis_index("s") * NUM_CORES + lax.axis_index("c")
    base = wid * rois_per_worker
    pltpu.sync_copy(roisp.at[pl.ds(base, rois_per_worker)], rois_v)

    io = lax.iota(jnp.int32, LANES)
    sel1 = io >> 3
    pf = (io & 7).astype(jnp.float32)
    limf = jnp.where(io < 8, float(w), float(h))
    limi = jnp.where(io < 8, w, h)
    j4 = io & 3
    kq = io >> 2

    def bounds(r):
        roi = rois_v[r]
        f1 = jnp.clip(_take16(roi, sel1) / 16.0, 0.0, limf - 1.0)
        f2 = jnp.clip(_take16(roi, sel1 + 2) / 16.0, f1 + 1.0, limf)
        bsz = (f2 - f1) / float(POOL)
        sf = f1 + pf * bsz
        ef = f1 + (pf + 1.0) * bsz
        s = jnp.maximum(sf.astype(jnp.int32), 0)
        ei = ef.astype(jnp.int32)
        e = jnp.minimum(jnp.where(ef > ei.astype(jnp.float32), ei + 1, ei),
                        limi)
        return s, e - 1

    def issue(r, idx_ref, patch_ref, sem):
        @pl.when(jnp.logical_and(r < rois_per_worker, base + r < n_rois))
        def _():
            s, _b = bounds(r)
            i0 = io * 0
            sy0 = _take16(s, i0 + 8)
            sx0 = _take16(s, i0)
            for v in range(4):
                k = io + 16 * v
                pi = k >> 3
                pj = k & 7
                idx_ref[pl.ds(v * 16, 16)] = (
                    jnp.minimum(sy0 + pi, h - 1) * w
                    + jnp.minimum(sx0 + pj, w - 1))
            pltpu.async_copy(table.at[idx_ref], patch_ref, sem)

    def compute(r, idx_ref, patch_ref, sem, out_ref, osem):
        n = base + r

        @pl.when(jnp.logical_and(r >= 2, n - 2 < n_rois))
        def _():
            pltpu.make_async_copy(out_ref, out.at[n - 2], osem).wait()

        @pl.when(n < n_rois)
        def _():
            s, b = bounds(r)
            i0 = io * 0
            base8 = _take16(s, i0 + 8) * 8 + _take16(s, i0)
            for v in range(13):
                k = jnp.minimum(4 * v + kq, POOL * POOL - 1)
                p_ = (k * 37) >> 8
                q_ = k - p_ * POOL
                ys = jnp.where(j4 < 2, _take16(s, p_ + 8),
                               _take16(b, p_ + 8)) * 8
                xs = jnp.where((j4 & 1) == 0, _take16(s, q_),
                               _take16(b, q_))
                off_all[pl.ds(v * 16, 16)] = ys + xs - base8
            pltpu.make_async_copy(table.at[idx_ref], patch_ref, sem).wait()

            def bin_body(kk, c2):
                vo = off_all[pl.ds(kk * 4, 16)]
                ra = vo[0]
                rb = vo[1]
                rc = vo[2]
                rd = vo[3]
                for c in range(0, 256, LANES):
                    sl = pl.ds(c, LANES)
                    m1 = jnp.maximum(patch_ref[ra, sl], patch_ref[rb, sl])
                    m2 = jnp.maximum(patch_ref[rc, sl], patch_ref[rd, sl])
                    out_ref[kk, sl] = jnp.maximum(m1, m2)
                return c2

            lax.fori_loop(0, POOL * POOL, bin_body, 0)
            pltpu.async_copy(out_ref, out.at[n], osem)

    issue(0, idx_p0, patch0, sem0)

    def pipe_body(g, carry):
        r0 = g * 2
        issue(r0 + 1, idx_p1, patch1, sem1)
        compute(r0, idx_p0, patch0, sem0, out_v0, osem0)
        issue(r0 + 2, idx_p0, patch0, sem0)
        compute(r0 + 1, idx_p1, patch1, sem1, out_v1, osem1)
        return carry

    lax.fori_loop(0, rois_per_worker // 2, pipe_body, 0)

    n0 = base + rois_per_worker - 2
    n1 = base + rois_per_worker - 1

    @pl.when(n0 < n_rois)
    def _():
        pltpu.make_async_copy(out_v0, out.at[n0], osem0).wait()

    @pl.when(n1 < n_rois)
    def _():
        pltpu.make_async_copy(out_v1, out.at[n1], osem1).wait()


@functools.lru_cache(maxsize=None)
def _build(n_rois, h, w, c):
    n_pad = -(-n_rois // NUM_WORKERS) * NUM_WORKERS
    rois_per_worker = n_pad // NUM_WORKERS
    mesh = plsc.VectorSubcoreMesh(core_axis_name="c", subcore_axis_name="s")
    body = functools.partial(_roi_pool_body, n_rois, h, w, rois_per_worker)
    return pl.kernel(
        body,
        mesh=mesh,
        out_type=jax.ShapeDtypeStruct((n_rois, POOL * POOL, c), jnp.float32),
        scratch_types=[
            pltpu.VMEM((rois_per_worker, LANES), jnp.float32),
            pltpu.VMEM((64,), jnp.int32),
            pltpu.VMEM((64,), jnp.int32),
            pltpu.VMEM((64, 256), jnp.float32),
            pltpu.VMEM((64, 256), jnp.float32),
            pltpu.VMEM((224,), jnp.int32),
            pltpu.VMEM((POOL * POOL, 256), jnp.float32),
            pltpu.VMEM((POOL * POOL, 256), jnp.float32),
            pltpu.SemaphoreType.DMA,
            pltpu.SemaphoreType.DMA,
            pltpu.SemaphoreType.DMA,
            pltpu.SemaphoreType.DMA,
        ],
    ), n_pad


def kernel(feat_map, rois):
    b, h, w, c = feat_map.shape
    n = rois.shape[1]
    fn, n_pad = _build(n, h, w, c)
    table = feat_map.reshape(h * w, c)
    roisp = jnp.zeros((n_pad, LANES), jnp.float32).at[:n, :4].set(
        rois.reshape(n, 4))
    out = fn(table, roisp)
    return out.reshape(b, n, POOL, POOL, c)
